# baseline (speedup 1.0000x reference)
import functools

import jax
import jax.numpy as jnp
from jax import lax
from jax.experimental import pallas as pl
from jax.experimental.pallas import tpu as pltpu
from jax.experimental.pallas import tpu_sc as plsc

D_MODEL_K = 64
SCALE_K = 8.0

_NC = 2
_NS = 16
_NW = _NC * _NS
_LANES = 16

_J = 200
_I = 4096
_P = 128


def _emb_kernel(tpair_hbm, xT_hbm, out_hbm, idx_v, pair_v, rb0, rb1,
                ob0, ob1, sg0, sg1, sw0, sw1):
    rbufs = (rb0, rb1)
    obufs = (ob0, ob1)
    sgs = (sg0, sg1)
    sws = (sw0, sw1)
    wid = lax.axis_index("s") * _NC + lax.axis_index("c")
    i0 = wid * _P
    pltpu.sync_copy(xT_hbm.at[:, pl.ds(i0, _P)], idx_v)

    dipre = [
        lax.iota(jnp.int32, _LANES) + dg * _LANES
        for dg in range(_P // _LANES)
    ]

    @plsc.parallel_loop(0, _J, unroll=2)
    def _(jrow):
        for dg in range(_P // _LANES):
            sl = pl.ds(dg * _LANES, _LANES)
            pair_v[jrow, sl] = lax.shift_right_logical(idx_v[jrow, sl], 1)

    def start_gather(j, b):
        pltpu.async_copy(tpair_hbm.at[pair_v.at[j]], rbufs[b], sgs[b])

    def wait_gather(j, b):
        pltpu.make_async_copy(tpair_hbm.at[pair_v.at[j]], rbufs[b],
                              sgs[b]).wait()

    def start_wb(j, b):
        pltpu.async_copy(obufs[b], out_hbm.at[j, :, pl.ds(i0, _P)], sws[b])

    def wait_wb(j, b):
        pltpu.make_async_copy(obufs[b], out_hbm.at[j, :, pl.ds(i0, _P)],
                              sws[b]).wait()

    start_gather(0, 0)
    start_gather(1, 1)

    def pair_step(jj, carry):
        for b in range(2):
            j = jj * 2 + b

            @pl.when(j >= 2)
            def _():
                wait_wb(j - 2, b)

            wait_gather(j, b)

            for dg in range(_P // _LANES):
                r16 = idx_v[j, pl.ds(dg * _LANES, _LANES)]
                half = lax.shift_left(
                    lax.bitwise_and(r16, jnp.int32(1)), 6)
                rows = dipre[dg]

                for k in range(D_MODEL_K):
                    col = half + k
                    vals = plsc.load_gather(rbufs[b], [rows, col])
                    obufs[b][k, pl.ds(dg * _LANES, _LANES)] = (
                        vals * SCALE_K)

            @pl.when(j + 2 < _J)
            def _():
                start_gather(j + 2, b)

            start_wb(j, b)
        return carry

    lax.fori_loop(0, _J // 2, pair_step, 0)
    wait_wb(_J - 2, 0)
    wait_wb(_J - 1, 1)


@jax.jit
def _emb_call(tpair, xT):
    mesh = plsc.VectorSubcoreMesh(core_axis_name="c", subcore_axis_name="s")
    run = functools.partial(
        pl.kernel,
        mesh=mesh,
        out_type=jax.ShapeDtypeStruct((_J, D_MODEL_K, _I), jnp.float32),
        compiler_params=pltpu.CompilerParams(
            use_tc_tiling_on_sc=True, needs_layout_passes=False),
        scratch_types=(
            [pltpu.VMEM((_J, _P), jnp.int32)] * 2
            + [pltpu.VMEM((_P, _P), jnp.float32)] * 2
            + [pltpu.VMEM((D_MODEL_K, _P), jnp.float32)] * 2
            + [pltpu.SemaphoreType.DMA] * 4
        ),
    )(_emb_kernel)
    return run(tpair, xT)


def kernel(x, table):
    xT = jnp.transpose(x).astype(jnp.int32)
    tpair = table.reshape(500000, _P)
    outT = _emb_call(tpair, xT)
    return jnp.transpose(outT, (2, 0, 1))


# speedup vs baseline: 1.6156x; 1.6156x over previous
import functools

import jax
import jax.numpy as jnp
from jax import lax
from jax.experimental import pallas as pl
from jax.experimental.pallas import tpu as pltpu
from jax.experimental.pallas import tpu_sc as plsc

D_MODEL_K = 64
SCALE_K = 8.0

_NC = 2
_NS = 16
_NW = _NC * _NS
_LANES = 16

_J = 200
_I = 4096
_P = 128
_NG = 4
_NO = 2


def _emb_kernel(tpair_hbm, xT_hbm, out_hbm, idx_v,
                pv0, pv1, pv2, pv3, rb0, rb1, rb2, rb3,
                ob0, ob1, sg0, sg1, sg2, sg3, sw0, sw1):
    pbufs = (pv0, pv1, pv2, pv3)
    rbufs = (rb0, rb1, rb2, rb3)
    obufs = (ob0, ob1)
    sgs = (sg0, sg1, sg2, sg3)
    sws = (sw0, sw1)
    wid = lax.axis_index("s") * _NC + lax.axis_index("c")
    i0 = wid * _P
    pltpu.sync_copy(xT_hbm.at[:, pl.ds(i0, _P)], idx_v)

    dipre = [
        lax.iota(jnp.int32, _LANES) + dg * _LANES
        for dg in range(_P // _LANES)
    ]

    def fill_and_gather(j, bg):
        for dg in range(_P // _LANES):
            sl = pl.ds(dg * _LANES, _LANES)
            pbufs[bg][sl] = lax.shift_right_logical(idx_v[j, sl], 1)
        pltpu.async_copy(tpair_hbm.at[pbufs[bg]], rbufs[bg], sgs[bg])

    def wait_gather(bg):
        pltpu.make_async_copy(tpair_hbm.at[pbufs[bg]], rbufs[bg],
                              sgs[bg]).wait()

    def start_wb(j, bo):
        pltpu.async_copy(obufs[bo], out_hbm.at[j, :, pl.ds(i0, _P)],
                         sws[bo])

    def wait_wb(j, bo):
        pltpu.make_async_copy(obufs[bo], out_hbm.at[j, :, pl.ds(i0, _P)],
                              sws[bo]).wait()

    for u in range(_NG):
        fill_and_gather(u, u)

    def step(jj, carry):
        for u in range(_NG):
            j = jj * _NG + u
            bo = u % _NO

            @pl.when(j >= _NO)
            def _():
                wait_wb(j - _NO, bo)

            wait_gather(u)

            for dg in range(_P // _LANES):
                r16 = idx_v[j, pl.ds(dg * _LANES, _LANES)]
                half = lax.shift_left(
                    lax.bitwise_and(r16, jnp.int32(1)), 6)
                rows = dipre[dg]

                @plsc.parallel_loop(0, D_MODEL_K, unroll=8)
                def _(k):
                    col = half + k
                    vals = plsc.load_gather(rbufs[u], [rows, col])
                    obufs[bo][k, pl.ds(dg * _LANES, _LANES)] = (
                        vals * SCALE_K)

            @pl.when(j + _NG < _J)
            def _():
                fill_and_gather(j + _NG, u)

            start_wb(j, bo)
        return carry

    lax.fori_loop(0, _J // _NG, step, 0)
    wait_wb(_J - 2, 0)
    wait_wb(_J - 1, 1)


@jax.jit
def _emb_call(tpair, xT):
    mesh = plsc.VectorSubcoreMesh(core_axis_name="c", subcore_axis_name="s")
    run = functools.partial(
        pl.kernel,
        mesh=mesh,
        out_type=jax.ShapeDtypeStruct((_J, D_MODEL_K, _I), jnp.float32),
        compiler_params=pltpu.CompilerParams(
            use_tc_tiling_on_sc=True, needs_layout_passes=False),
        scratch_types=(
            [pltpu.VMEM((_J, _P), jnp.int32)]
            + [pltpu.VMEM((_P,), jnp.int32)] * _NG
            + [pltpu.VMEM((_P, _P), jnp.float32)] * _NG
            + [pltpu.VMEM((D_MODEL_K, _P), jnp.float32)] * _NO
            + [pltpu.SemaphoreType.DMA] * (_NG + _NO)
        ),
    )(_emb_kernel)
    return run(tpair, xT)


def kernel(x, table):
    xT = jnp.transpose(x).astype(jnp.int32)
    tpair = table.reshape(500000, _P)
    outT = _emb_call(tpair, xT)
    return jnp.transpose(outT, (2, 0, 1))
